# baseline (speedup 1.0000x reference)
import jax
import jax.numpy as jnp
from jax import lax
from jax.experimental import pallas as pl


def _dkvb_body(xh_ref, xn_ref, ke_ref, kn_ref, val_ref, out_ref):
    tn = xh_ref.shape[1]
    c = ke_ref.shape[1]
    xb = xh_ref[0]
    ke = ke_ref[0]
    xn = xn_ref[0]
    kn = kn_ref[0]
    dot = lax.dot_general(xb, ke, (((1,), (1,)), ((), ())),
                          preferred_element_type=jnp.float32)
    d2 = (xn - 2.0 * dot) + kn
    dist = -jnp.sqrt(jnp.maximum(d2, 0.0))
    m = jnp.max(dist, axis=1, keepdims=True)
    iota = lax.broadcasted_iota(jnp.int32, (tn, c), 1)
    idx = jnp.min(jnp.where(dist == m, iota, c), axis=1)
    onehot = (iota == idx[:, None]).astype(jnp.float32)
    out_ref[0] = lax.dot_general(onehot, val_ref[0], (((1,), (0,)), ((), ())),
                                 precision=lax.Precision.HIGHEST,
                                 preferred_element_type=jnp.float32)


def kernel(x, mask, key_embed, values, key_optim):
    x = x.astype(jnp.float32)
    b, t, dim = x.shape
    h, c, d = key_embed.shape
    dv = values.shape[-1]
    n = b * t
    tn = 512

    xh = jnp.transpose(x.reshape(b, t, h, d), (2, 0, 1, 3)).reshape(h, n, d)
    xn = jnp.sum(xh ** 2, axis=-1, keepdims=True)
    kn = jnp.sum(key_embed ** 2, axis=-1)[:, None, :]

    out = pl.pallas_call(
        _dkvb_body,
        grid=(h, n // tn),
        in_specs=[
            pl.BlockSpec((1, tn, d), lambda hh, i: (hh, i, 0)),
            pl.BlockSpec((1, tn, 1), lambda hh, i: (hh, i, 0)),
            pl.BlockSpec((1, c, d), lambda hh, i: (hh, 0, 0)),
            pl.BlockSpec((1, 1, c), lambda hh, i: (hh, 0, 0)),
            pl.BlockSpec((1, c, dv), lambda hh, i: (hh, 0, 0)),
        ],
        out_specs=pl.BlockSpec((1, tn, dv), lambda hh, i: (hh, i, 0)),
        out_shape=jax.ShapeDtypeStruct((h, n, dv), jnp.float32),
    )(xh, xn, key_embed, kn, values)

    return jnp.transpose(out, (1, 0, 2)).reshape(b, t, h * dv)


# speedup vs baseline: 1.8755x; 1.8755x over previous
import jax
import jax.numpy as jnp
from jax import lax
from jax.experimental import pallas as pl

_TN = 512


def _dkvb_body(xh_ref, xn_ref, ke_ref, kn_ref, val_ref, out_ref):
    tn = xh_ref.shape[1]
    c = ke_ref.shape[1]
    dot = lax.dot_general(xh_ref[0], ke_ref[0], (((1,), (1,)), ((), ())),
                          preferred_element_type=jnp.float32)
    d2 = (xn_ref[0] - 2.0 * dot) + kn_ref[0]
    rmin = jnp.min(d2, axis=1, keepdims=True)
    s = jnp.sqrt(jnp.maximum(rmin, 0.0))
    s_next = lax.bitcast_convert_type(
        lax.bitcast_convert_type(s, jnp.int32) + 1, jnp.float32)
    t0 = s * s_next
    t0_pred = lax.bitcast_convert_type(
        lax.bitcast_convert_type(t0, jnp.int32) - 1, jnp.float32)
    tle = jnp.where(jnp.sqrt(t0) == s, t0, t0_pred)
    thr = jnp.where(rmin > 0.0, jnp.maximum(tle, rmin), 0.0)
    iota = lax.broadcasted_iota(jnp.int32, (tn, c), 1)
    sel = jnp.where(d2 <= thr, iota, c)
    idx = jnp.min(sel, axis=1, keepdims=True)
    onehot = (sel == idx).astype(jnp.float32)
    out_ref[0] = lax.dot_general(onehot, val_ref[0], (((1,), (0,)), ((), ())),
                                 preferred_element_type=jnp.float32)


def kernel(x, mask, key_embed, values, key_optim):
    x = x.astype(jnp.float32)
    b, t, dim = x.shape
    h, c, d = key_embed.shape
    dv = values.shape[-1]
    n = b * t
    tn = _TN

    xh = jnp.transpose(x.reshape(b, t, h, d), (2, 0, 1, 3)).reshape(h, n, d)
    xn = jnp.sum(xh ** 2, axis=-1, keepdims=True)
    kn = jnp.sum(key_embed ** 2, axis=-1)[:, None, :]

    out = pl.pallas_call(
        _dkvb_body,
        grid=(h, n // tn),
        in_specs=[
            pl.BlockSpec((1, tn, d), lambda hh, i: (hh, i, 0)),
            pl.BlockSpec((1, tn, 1), lambda hh, i: (hh, i, 0)),
            pl.BlockSpec((1, c, d), lambda hh, i: (hh, 0, 0)),
            pl.BlockSpec((1, 1, c), lambda hh, i: (hh, 0, 0)),
            pl.BlockSpec((1, c, dv), lambda hh, i: (hh, 0, 0)),
        ],
        out_specs=pl.BlockSpec((1, tn, dv), lambda hh, i: (hh, i, 0)),
        out_shape=jax.ShapeDtypeStruct((h, n, dv), jnp.float32),
    )(xh, xn, key_embed, kn, values)

    return jnp.transpose(out, (1, 0, 2)).reshape(b, t, h * dv)
